# baseline (speedup 1.0000x reference)
import functools

import jax
import jax.numpy as jnp
from jax import lax
from jax.experimental import pallas as pl
from jax.experimental.pallas import tpu as pltpu
from jax.experimental.pallas import tpu_sc as plsc

N = 10000
E = 320000
D = 128
C = 129

NC = 2
NS = 16
NW = NC * NS

N_PAD = 10240
DUMMY = N
K = 128
NCH0 = 112
NCH1 = 48
NCHM = NCH0
E_PAD = NS * K * (NCH0 + NCH1)

BN = 512
GRID = N_PAD // BN

_mesh = plsc.VectorSubcoreMesh(core_axis_name="c", subcore_axis_name="s")


def _wid():
    return lax.axis_index("s") * NC + lax.axis_index("c")


@functools.partial(
    pl.kernel,
    out_type=[
        jax.ShapeDtypeStruct((NC, N_PAD), jnp.float32),
        jax.ShapeDtypeStruct((NC, N_PAD), jnp.float32),
    ],
    mesh=_mesh,
    scratch_types=[
        pltpu.VMEM((NCHM, K), jnp.int32),
        pltpu.VMEM((NCHM, K), jnp.int32),
        pltpu.VMEM((K,), jnp.float32),
        pltpu.VMEM((N_PAD // NS,), jnp.float32),
        pltpu.VMEM_SHARED((N_PAD,), jnp.float32),
        pltpu.VMEM_SHARED((N_PAD,), jnp.float32),
    ],
)
def _deg_kernel(src_hbm, dst_hbm, outs_hbm, outd_hbm, idx_s, idx_d, ones_v,
                zwin, acc_s, acc_d):
    c = lax.axis_index("c")
    s = lax.axis_index("s")
    cnt = jnp.where(c == 0, NCH0, NCH1)
    rpt = N_PAD // NS

    pltpu.sync_copy(src_hbm.at[c, s], idx_s)
    pltpu.sync_copy(dst_hbm.at[c, s], idx_d)

    ones = jnp.ones((16,), jnp.float32)
    zeros = jnp.zeros((16,), jnp.float32)

    def o_body(i, _):
        ones_v[pl.ds(i * 16, 16)] = ones
        return 0

    lax.fori_loop(0, K // 16, o_body, 0)

    def z_body(i, _):
        zwin[pl.ds(i * 16, 16)] = zeros
        return 0

    lax.fori_loop(0, rpt // 16, z_body, 0)
    pltpu.sync_copy(zwin, acc_s.at[pl.ds(s * rpt, rpt)])
    pltpu.sync_copy(zwin, acc_d.at[pl.ds(s * rpt, rpt)])
    plsc.subcore_barrier()

    def e_body(j, _):
        pltpu.sync_copy(ones_v, acc_s.at[idx_s.at[j]], add=True)
        pltpu.sync_copy(ones_v, acc_d.at[idx_d.at[j]], add=True)
        return 0

    lax.fori_loop(0, cnt, e_body, 0)
    plsc.subcore_barrier()

    pltpu.sync_copy(acc_s.at[pl.ds(s * rpt, rpt)], outs_hbm.at[c, pl.ds(s * rpt, rpt)])
    pltpu.sync_copy(acc_d.at[pl.ds(s * rpt, rpt)], outd_hbm.at[c, pl.ds(s * rpt, rpt)])


@functools.partial(
    pl.kernel,
    out_type=jax.ShapeDtypeStruct((NC, N_PAD, D), jnp.float32),
    mesh=_mesh,
    scratch_types=[
        pltpu.VMEM((4, K), jnp.int32),
        pltpu.VMEM((4, K), jnp.int32),
        pltpu.VMEM((K, D), jnp.float32),
        pltpu.VMEM((K, D), jnp.float32),
        pltpu.VMEM_SHARED((N_PAD, D), jnp.float32),
        [pltpu.SemaphoreType.DMA] * 2,
        [pltpu.SemaphoreType.DMA] * 2,
        [pltpu.SemaphoreType.DMA] * 4,
        [pltpu.SemaphoreType.DMA] * 4,
    ],
)
def _spmm_kernel(h_hbm, src_hbm, dst_hbm, out_hbm, idx_s, idx_d, bufa, bufb,
                 acc, gsem, ssem, isem, idsem):
    c = lax.axis_index("c")
    s = lax.axis_index("s")
    cnt = jnp.where(c == 0, NCH0, NCH1)
    rows_per_tile = N_PAD // NS
    bufs = (bufa, bufb)

    zeros = jnp.zeros((16,), jnp.float32)

    def z_body(i, _):
        bufa[i // (D // 16), pl.ds((i % (D // 16)) * 16, 16)] = zeros
        return 0

    lax.fori_loop(0, K * D // 16, z_body, 0)

    def zc_body(i, _):
        pltpu.sync_copy(bufa, acc.at[pl.ds(s * rows_per_tile + i * K, K)])
        return 0

    lax.fori_loop(0, rows_per_tile // K, zc_body, 0)
    plsc.subcore_barrier()

    def ifetch(j, slot):
        pltpu.async_copy(src_hbm.at[c, s, j], idx_s.at[slot], isem[slot])
        pltpu.async_copy(dst_hbm.at[c, s, j], idx_d.at[slot], idsem[slot])

    def i_wait(slot):
        pltpu.make_async_copy(src_hbm.at[0, 0, 0], idx_s.at[slot],
                              isem[slot]).wait()
        pltpu.make_async_copy(src_hbm.at[0, 0, 0], idx_d.at[slot],
                              idsem[slot]).wait()

    def g_start(slot, u):
        pltpu.async_copy(h_hbm.at[idx_s.at[slot]], bufs[u % 2], gsem[u % 2])

    def g_wait(u):
        pltpu.make_async_copy(h_hbm.at[pl.ds(0, K)], bufs[u % 2],
                              gsem[u % 2]).wait()

    def s_start(slot, u):
        pltpu.async_copy(bufs[u % 2], acc.at[idx_d.at[slot]], ssem[u % 2],
                         add=True)

    def s_wait(u):
        pltpu.make_async_copy(h_hbm.at[pl.ds(0, K)], bufs[u % 2],
                              ssem[u % 2]).wait()

    rounds = cnt // 4
    for slot in range(3):
        ifetch(slot, slot)
    i_wait(0)
    g_start(0, 0)

    def body(i, _):
        a = 4 * i
        for u in range(4):
            g_wait(u)
            if u < 3:
                i_wait(u + 1)
                g_start(u + 1, u + 1)
            else:
                @pl.when(i + 1 < rounds)
                def _():
                    i_wait(0)
                    g_start(0, 0)
            s_start(u, u)

            @pl.when(a + u + 3 < cnt)
            def _():
                ifetch(a + u + 3, (u + 3) % 4)

            s_wait(u)
        return 0

    lax.fori_loop(0, rounds, body, 0)
    plsc.subcore_barrier()

    pltpu.sync_copy(
        acc.at[pl.ds(s * rows_per_tile, rows_per_tile)],
        out_hbm.at[c, pl.ds(s * rows_per_tile, rows_per_tile)],
    )


def _p0_body(x_ref, degs_ref, degd_ref, wfc_ref, bfc_ref,
             h1_ref, g_ref, ns_ref, nd_ref):
    i = pl.program_id(0)
    ones32 = jnp.ones((NC, 1), jnp.float32)
    ds_col = lax.dot_general(degs_ref[...], ones32, (((0,), (0,)), ((), ())))
    dd_col = lax.dot_general(degd_ref[...], ones32, (((0,), (0,)), ((), ())))
    row = lax.broadcasted_iota(jnp.int32, (BN, 1), 0) + i * BN
    valid = row < N
    ns = jnp.where(valid, lax.rsqrt(jnp.clip(ds_col, 1.0, None)), 0.0)
    nd = jnp.where(valid, lax.rsqrt(jnp.clip(dd_col, 1.0, None)), 0.0)
    g = jnp.dot(x_ref[...], wfc_ref[...], preferred_element_type=jnp.float32,
                precision=lax.Precision.HIGHEST) + bfc_ref[...]
    h1_ref[...] = g * ns
    g_ref[...] = g
    ns_ref[...] = ns
    nd_ref[...] = nd


def _p0(x, degs, degd, wfc, bfc):
    return pl.pallas_call(
        _p0_body,
        grid=(GRID,),
        in_specs=[
            pl.BlockSpec((BN, D), lambda i: (i, 0)),
            pl.BlockSpec((NC, BN), lambda i: (0, i)),
            pl.BlockSpec((NC, BN), lambda i: (0, i)),
            pl.BlockSpec((D, D), lambda i: (0, 0)),
            pl.BlockSpec((1, D), lambda i: (0, 0)),
        ],
        out_specs=[
            pl.BlockSpec((BN, D), lambda i: (i, 0)),
            pl.BlockSpec((BN, D), lambda i: (i, 0)),
            pl.BlockSpec((BN, 1), lambda i: (i, 0)),
            pl.BlockSpec((BN, 1), lambda i: (i, 0)),
        ],
        out_shape=[
            jax.ShapeDtypeStruct((N_PAD, D), jnp.float32),
            jax.ShapeDtypeStruct((N_PAD, D), jnp.float32),
            jax.ShapeDtypeStruct((N_PAD, 1), jnp.float32),
            jax.ShapeDtypeStruct((N_PAD, 1), jnp.float32),
        ],
    )(x, degs, degd, wfc, bfc)


def _leaky(x):
    return jnp.where(x >= 0, x, 0.01 * x)


def _pg_body(parts_ref, nd_ref, ns_ref, prev_ref, w_ref, b_ref, wg_ref, bg_ref,
             g_out_ref, h_out_ref):
    agg = (parts_ref[0] + parts_ref[1]) * nd_ref[...]
    x = _leaky(jnp.dot(agg, w_ref[...], preferred_element_type=jnp.float32,
                       precision=lax.Precision.HIGHEST) + b_ref[...])
    t = jnp.dot(x, wg_ref[...], preferred_element_type=jnp.float32,
                precision=lax.Precision.HIGHEST) + bg_ref[...]
    g = x * t + prev_ref[...] * (1.0 - t)
    g_out_ref[...] = g
    h_out_ref[...] = g * ns_ref[...]


def _pg(parts, nd, ns, prev, w, b, wg, bg):
    return pl.pallas_call(
        _pg_body,
        grid=(GRID,),
        in_specs=[
            pl.BlockSpec((NC, BN, D), lambda i: (0, i, 0)),
            pl.BlockSpec((BN, 1), lambda i: (i, 0)),
            pl.BlockSpec((BN, 1), lambda i: (i, 0)),
            pl.BlockSpec((BN, D), lambda i: (i, 0)),
            pl.BlockSpec((D, D), lambda i: (0, 0)),
            pl.BlockSpec((1, D), lambda i: (0, 0)),
            pl.BlockSpec((D, 1), lambda i: (0, 0)),
            pl.BlockSpec((1, 1), lambda i: (0, 0)),
        ],
        out_specs=[
            pl.BlockSpec((BN, D), lambda i: (i, 0)),
            pl.BlockSpec((BN, D), lambda i: (i, 0)),
        ],
        out_shape=[
            jax.ShapeDtypeStruct((N_PAD, D), jnp.float32),
            jax.ShapeDtypeStruct((N_PAD, D), jnp.float32),
        ],
    )(parts, nd, ns, prev, w, b, wg, bg)


def _p3_body(parts_ref, nd_ref, w_ref, b_ref, out_ref):
    agg = (parts_ref[0] + parts_ref[1]) * nd_ref[...]
    out_ref[...] = _leaky(
        jnp.dot(agg, w_ref[...], preferred_element_type=jnp.float32,
                precision=lax.Precision.HIGHEST) + b_ref[...])


def _p3(parts, nd, w3p, b3p):
    return pl.pallas_call(
        _p3_body,
        grid=(GRID,),
        in_specs=[
            pl.BlockSpec((NC, BN, D), lambda i: (0, i, 0)),
            pl.BlockSpec((BN, 1), lambda i: (i, 0)),
            pl.BlockSpec((D, 256), lambda i: (0, 0)),
            pl.BlockSpec((1, 256), lambda i: (0, 0)),
        ],
        out_specs=pl.BlockSpec((BN, 256), lambda i: (i, 0)),
        out_shape=jax.ShapeDtypeStruct((N_PAD, 256), jnp.float32),
    )(parts, nd, w3p, b3p)


def kernel(g_init_emb, edge_index, Wfc, bfc, W1, W2, b2, W3, b3, Wg1, bg1, Wg2, bg2):
    src = edge_index[0]
    dst = edge_index[1]

    def pack(e):
        e = jnp.concatenate([e, jnp.full((E_PAD - E,), DUMMY, jnp.int32)])
        e0 = e[: NS * NCH0 * K].reshape(1, NS, NCH0, K)
        e1 = e[NS * NCH0 * K:].reshape(1, NS, NCH1, K)
        e1 = jnp.pad(e1, ((0, 0), (0, 0), (0, NCHM - NCH1), (0, 0)),
                     constant_values=DUMMY)
        return jnp.concatenate([e0, e1], axis=0)

    src_r = pack(src)
    dst_r = pack(dst)

    x = jnp.pad(g_init_emb, ((0, N_PAD - N), (0, 0)))
    w3p = jnp.pad(W3, ((0, 0), (0, 256 - C)))
    b3p = jnp.pad(b3, ((0, 256 - C),)).reshape(1, 256)

    degs, degd = _deg_kernel(src_r, dst_r)
    h1, g_feat, ns, nd = _p0(x, degs, degd, Wfc, bfc.reshape(1, D))

    parts1 = _spmm_kernel(h1, src_r, dst_r)
    g1, h2 = _pg(parts1, nd, ns, g_feat, W1, jnp.zeros((1, D), jnp.float32),
                 Wg1, bg1.reshape(1, 1))

    parts2 = _spmm_kernel(h2, src_r, dst_r)
    g2, h3 = _pg(parts2, nd, ns, g1, W2, b2.reshape(1, D), Wg2, bg2.reshape(1, 1))

    parts3 = _spmm_kernel(h3, src_r, dst_r)
    out = _p3(parts3, nd, w3p, b3p)
    return out[:N, :C]


# speedup vs baseline: 1.5659x; 1.5659x over previous
import functools

import jax
import jax.numpy as jnp
from jax import lax
from jax.experimental import pallas as pl
from jax.experimental.pallas import tpu as pltpu
from jax.experimental.pallas import tpu_sc as plsc

N = 10000
E = 320000
D = 128
C = 129

NC = 2
NS = 16
NW = NC * NS

N_PAD = 10240
DUMMY = N
K = 128
NCH0 = 112
NCH1 = 46
NCHM = NCH0
E_PAD = NS * K * (NCH0 + NCH1)

BN = 1024
GRID = N_PAD // BN

_mesh = plsc.VectorSubcoreMesh(core_axis_name="c", subcore_axis_name="s")


def _wid():
    return lax.axis_index("s") * NC + lax.axis_index("c")


@functools.partial(
    pl.kernel,
    out_type=[
        jax.ShapeDtypeStruct((NC, N_PAD), jnp.float32),
        jax.ShapeDtypeStruct((NC, N_PAD), jnp.float32),
    ],
    mesh=_mesh,
    scratch_types=[
        pltpu.VMEM((NCHM, K), jnp.int32),
        pltpu.VMEM((NCHM, K), jnp.int32),
        pltpu.VMEM((K,), jnp.float32),
        pltpu.VMEM((N_PAD // NS,), jnp.float32),
        pltpu.VMEM_SHARED((N_PAD,), jnp.float32),
        pltpu.VMEM_SHARED((N_PAD,), jnp.float32),
    ],
)
def _deg_kernel(src_hbm, dst_hbm, outs_hbm, outd_hbm,
                idx_s, idx_d, ones_v, zwin, acc_s, acc_d):
    c = lax.axis_index("c")
    s = lax.axis_index("s")
    cnt = jnp.where(c == 0, NCH0, NCH1)
    rpt = N_PAD // NS

    pltpu.sync_copy(src_hbm.at[c, s], idx_s)
    pltpu.sync_copy(dst_hbm.at[c, s], idx_d)

    ones = jnp.ones((16,), jnp.float32)
    zeros = jnp.zeros((16,), jnp.float32)

    def o_body(i, _):
        ones_v[pl.ds(i * 16, 16)] = ones
        return 0

    lax.fori_loop(0, K // 16, o_body, 0)

    def z_body(i, _):
        zwin[pl.ds(i * 16, 16)] = zeros
        return 0

    lax.fori_loop(0, rpt // 16, z_body, 0)
    pltpu.sync_copy(zwin, acc_s.at[pl.ds(s * rpt, rpt)])
    pltpu.sync_copy(zwin, acc_d.at[pl.ds(s * rpt, rpt)])
    plsc.subcore_barrier()

    def e_body(j, _):
        pltpu.sync_copy(ones_v, acc_s.at[idx_s.at[j]], add=True)
        pltpu.sync_copy(ones_v, acc_d.at[idx_d.at[j]], add=True)
        return 0

    lax.fori_loop(0, cnt, e_body, 0)
    plsc.subcore_barrier()

    pltpu.sync_copy(acc_s.at[pl.ds(s * rpt, rpt)], outs_hbm.at[c, pl.ds(s * rpt, rpt)])
    pltpu.sync_copy(acc_d.at[pl.ds(s * rpt, rpt)], outd_hbm.at[c, pl.ds(s * rpt, rpt)])


@functools.partial(
    pl.kernel,
    out_type=jax.ShapeDtypeStruct((NC, N_PAD, D), jnp.float32),
    mesh=_mesh,
    scratch_types=[
        pltpu.VMEM((NCHM, K), jnp.int32),
        pltpu.VMEM((NCHM, K), jnp.int32),
        pltpu.VMEM((K, D), jnp.float32),
        pltpu.VMEM_SHARED((N_PAD, D), jnp.float32),
        pltpu.SemaphoreType.DMA,
    ],
)
def _spmm_kernel(h_hbm, src_hbm, dst_hbm, out_hbm,
                 idx_s, idx_d, gbuf, acc, sem):
    c = lax.axis_index("c")
    s = lax.axis_index("s")
    cnt = jnp.where(c == 0, NCH0, NCH1)
    rows_per_tile = N_PAD // NS

    pltpu.sync_copy(src_hbm.at[c, s], idx_s)
    pltpu.sync_copy(dst_hbm.at[c, s], idx_d)

    zeros = jnp.zeros((16,), jnp.float32)

    def z_body(i, _):
        gbuf[i // (D // 16), pl.ds((i % (D // 16)) * 16, 16)] = zeros
        return 0

    lax.fori_loop(0, K * D // 16, z_body, 0)

    def zc_body(i, _):
        pltpu.sync_copy(gbuf, acc.at[pl.ds(s * rows_per_tile + i * K, K)])
        return 0

    lax.fori_loop(0, rows_per_tile // K, zc_body, 0)
    plsc.subcore_barrier()

    def body(j, _):
        pltpu.async_copy(h_hbm.at[idx_s.at[j]], gbuf, sem).wait()
        pltpu.sync_copy(gbuf, acc.at[idx_d.at[j]], add=True)
        return 0

    lax.fori_loop(0, cnt, body, 0)
    plsc.subcore_barrier()

    pltpu.sync_copy(
        acc.at[pl.ds(s * rows_per_tile, rows_per_tile)],
        out_hbm.at[c, pl.ds(s * rows_per_tile, rows_per_tile)],
    )


def _p0a_body(x_ref, wfc_ref, bfc_ref, g_ref):
    g_ref[...] = jnp.dot(x_ref[...], wfc_ref[...],
                         preferred_element_type=jnp.float32) + bfc_ref[...]


def _p0a(x, wfc, bfc):
    return pl.pallas_call(
        _p0a_body,
        grid=(GRID,),
        in_specs=[
            pl.BlockSpec((BN, D), lambda i: (i, 0)),
            pl.BlockSpec((D, D), lambda i: (0, 0)),
            pl.BlockSpec((1, D), lambda i: (0, 0)),
        ],
        out_specs=pl.BlockSpec((BN, D), lambda i: (i, 0)),
        out_shape=jax.ShapeDtypeStruct((N_PAD, D), jnp.float32),
    )(x, wfc, bfc)


def _p0b_body(g_ref, degs_ref, degd_ref, h1_ref, ns_ref, nd_ref):
    i = pl.program_id(0)
    ones2 = jnp.ones((NC, 1), jnp.float32)
    ds_col = lax.dot_general(degs_ref[...], ones2, (((0,), (0,)), ((), ())))
    dd_col = lax.dot_general(degd_ref[...], ones2, (((0,), (0,)), ((), ())))
    row = lax.broadcasted_iota(jnp.int32, (BN, 1), 0) + i * BN
    valid = row < N
    ns = jnp.where(valid, lax.rsqrt(jnp.clip(ds_col, 1.0, None)), 0.0)
    nd = jnp.where(valid, lax.rsqrt(jnp.clip(dd_col, 1.0, None)), 0.0)
    h1_ref[...] = g_ref[...] * ns
    ns_ref[...] = ns
    nd_ref[...] = nd


def _p0b(g, degs, degd):
    return pl.pallas_call(
        _p0b_body,
        grid=(GRID,),
        in_specs=[
            pl.BlockSpec((BN, D), lambda i: (i, 0)),
            pl.BlockSpec((NC, BN), lambda i: (0, i)),
            pl.BlockSpec((NC, BN), lambda i: (0, i)),
        ],
        out_specs=[
            pl.BlockSpec((BN, D), lambda i: (i, 0)),
            pl.BlockSpec((BN, 1), lambda i: (i, 0)),
            pl.BlockSpec((BN, 1), lambda i: (i, 0)),
        ],
        out_shape=[
            jax.ShapeDtypeStruct((N_PAD, D), jnp.float32),
            jax.ShapeDtypeStruct((N_PAD, 1), jnp.float32),
            jax.ShapeDtypeStruct((N_PAD, 1), jnp.float32),
        ],
    )(g, degs, degd)


def _leaky(x):
    return jnp.where(x >= 0, x, 0.01 * x)


def _pg_body(parts_ref, nd_ref, ns_ref, prev_ref, w_ref, b_ref, wg_ref, bg_ref,
             g_out_ref, h_out_ref):
    agg = (parts_ref[0] + parts_ref[1]) * nd_ref[...]
    x = _leaky(jnp.dot(agg, w_ref[...],
                       preferred_element_type=jnp.float32) + b_ref[...])
    t = jnp.dot(x, wg_ref[...], preferred_element_type=jnp.float32) + bg_ref[...]
    g = x * t + prev_ref[...] * (1.0 - t)
    g_out_ref[...] = g
    h_out_ref[...] = g * ns_ref[...]


def _pg(parts, nd, ns, prev, w, b, wg, bg):
    return pl.pallas_call(
        _pg_body,
        grid=(GRID,),
        in_specs=[
            pl.BlockSpec((NC, BN, D), lambda i: (0, i, 0)),
            pl.BlockSpec((BN, 1), lambda i: (i, 0)),
            pl.BlockSpec((BN, 1), lambda i: (i, 0)),
            pl.BlockSpec((BN, D), lambda i: (i, 0)),
            pl.BlockSpec((D, D), lambda i: (0, 0)),
            pl.BlockSpec((1, D), lambda i: (0, 0)),
            pl.BlockSpec((D, 1), lambda i: (0, 0)),
            pl.BlockSpec((1, 1), lambda i: (0, 0)),
        ],
        out_specs=[
            pl.BlockSpec((BN, D), lambda i: (i, 0)),
            pl.BlockSpec((BN, D), lambda i: (i, 0)),
        ],
        out_shape=[
            jax.ShapeDtypeStruct((N_PAD, D), jnp.float32),
            jax.ShapeDtypeStruct((N_PAD, D), jnp.float32),
        ],
    )(parts, nd, ns, prev, w, b, wg, bg)


def _p3_body(parts_ref, nd_ref, w_ref, b_ref, out_ref):
    agg = (parts_ref[0] + parts_ref[1]) * nd_ref[...]
    out_ref[...] = _leaky(
        jnp.dot(agg, w_ref[...],
                preferred_element_type=jnp.float32) + b_ref[...])


def _p3(parts, nd, w3p, b3p):
    return pl.pallas_call(
        _p3_body,
        grid=(GRID,),
        in_specs=[
            pl.BlockSpec((NC, BN, D), lambda i: (0, i, 0)),
            pl.BlockSpec((BN, 1), lambda i: (i, 0)),
            pl.BlockSpec((D, 256), lambda i: (0, 0)),
            pl.BlockSpec((1, 256), lambda i: (0, 0)),
        ],
        out_specs=pl.BlockSpec((BN, 256), lambda i: (i, 0)),
        out_shape=jax.ShapeDtypeStruct((N_PAD, 256), jnp.float32),
    )(parts, nd, w3p, b3p)


def kernel(g_init_emb, edge_index, Wfc, bfc, W1, W2, b2, W3, b3, Wg1, bg1, Wg2, bg2):
    src = edge_index[0]
    dst = edge_index[1]

    def pack(e):
        e = jnp.concatenate([e, jnp.full((E_PAD - E,), DUMMY, jnp.int32)])
        e0 = e[: NS * NCH0 * K].reshape(1, NS, NCH0, K)
        e1 = e[NS * NCH0 * K:].reshape(1, NS, NCH1, K)
        e1 = jnp.pad(e1, ((0, 0), (0, 0), (0, NCHM - NCH1), (0, 0)),
                     constant_values=DUMMY)
        return jnp.concatenate([e0, e1], axis=0)

    src_r = pack(src)
    dst_r = pack(dst)

    x = jnp.pad(g_init_emb, ((0, N_PAD - N), (0, 0)))
    w3p = jnp.pad(W3, ((0, 0), (0, 256 - C)))
    b3p = jnp.pad(b3, ((0, 256 - C),)).reshape(1, 256)

    degs, degd = _deg_kernel(src_r, dst_r)
    g_feat = _p0a(x, Wfc, bfc.reshape(1, D))
    h1, ns, nd = _p0b(g_feat, degs, degd)

    parts1 = _spmm_kernel(h1, src_r, dst_r)
    g1, h2 = _pg(parts1, nd, ns, g_feat, W1, jnp.zeros((1, D), jnp.float32),
                 Wg1, bg1.reshape(1, 1))

    parts2 = _spmm_kernel(h2, src_r, dst_r)
    g2, h3 = _pg(parts2, nd, ns, g1, W2, b2.reshape(1, D), Wg2, bg2.reshape(1, 1))

    parts3 = _spmm_kernel(h3, src_r, dst_r)
    out = _p3(parts3, nd, w3p, b3p)
    return out[:N, :C]


# speedup vs baseline: 1.5728x; 1.0045x over previous
import functools

import jax
import jax.numpy as jnp
from jax import lax
from jax.experimental import pallas as pl
from jax.experimental.pallas import tpu as pltpu
from jax.experimental.pallas import tpu_sc as plsc

N = 10000
E = 320000
D = 128
C = 129

NC = 2
NS = 16
NW = NC * NS

N_PAD = 10240
DUMMY = N
K = 128
NCH0 = 112
NCH1 = 46
NCHM = NCH0
E_PAD = NS * K * (NCH0 + NCH1)

BN = 1024
GRID = N_PAD // BN

_mesh = plsc.VectorSubcoreMesh(core_axis_name="c", subcore_axis_name="s")


def _wid():
    return lax.axis_index("s") * NC + lax.axis_index("c")


@functools.partial(
    pl.kernel,
    out_type=[
        jax.ShapeDtypeStruct((NC, N_PAD), jnp.float32),
        jax.ShapeDtypeStruct((NC, N_PAD), jnp.float32),
    ],
    mesh=_mesh,
    scratch_types=[
        pltpu.VMEM((NCHM, K), jnp.int32),
        pltpu.VMEM((NCHM, K), jnp.int32),
        pltpu.VMEM((K,), jnp.float32),
        pltpu.VMEM((N_PAD // NS,), jnp.float32),
        pltpu.VMEM_SHARED((N_PAD,), jnp.float32),
        pltpu.VMEM_SHARED((N_PAD,), jnp.float32),
    ],
)
def _deg_kernel(src_hbm, dst_hbm, outs_hbm, outd_hbm,
                idx_s, idx_d, ones_v, zwin, acc_s, acc_d):
    c = lax.axis_index("c")
    s = lax.axis_index("s")
    cnt = jnp.where(c == 0, NCH0, NCH1)
    rpt = N_PAD // NS

    pltpu.sync_copy(src_hbm.at[c, s], idx_s)
    pltpu.sync_copy(dst_hbm.at[c, s], idx_d)

    ones = jnp.ones((16,), jnp.float32)
    zeros = jnp.zeros((16,), jnp.float32)

    def o_body(i, _):
        ones_v[pl.ds(i * 16, 16)] = ones
        return 0

    lax.fori_loop(0, K // 16, o_body, 0)

    def z_body(i, _):
        zwin[pl.ds(i * 16, 16)] = zeros
        return 0

    lax.fori_loop(0, rpt // 16, z_body, 0)
    pltpu.sync_copy(zwin, acc_s.at[pl.ds(s * rpt, rpt)])
    pltpu.sync_copy(zwin, acc_d.at[pl.ds(s * rpt, rpt)])
    plsc.subcore_barrier()

    def e_body(j, _):
        pltpu.sync_copy(ones_v, acc_s.at[idx_s.at[j]], add=True)
        pltpu.sync_copy(ones_v, acc_d.at[idx_d.at[j]], add=True)
        return 0

    lax.fori_loop(0, cnt, e_body, 0)
    plsc.subcore_barrier()

    pltpu.sync_copy(acc_s.at[pl.ds(s * rpt, rpt)], outs_hbm.at[c, pl.ds(s * rpt, rpt)])
    pltpu.sync_copy(acc_d.at[pl.ds(s * rpt, rpt)], outd_hbm.at[c, pl.ds(s * rpt, rpt)])


@functools.partial(
    pl.kernel,
    out_type=jax.ShapeDtypeStruct((NC, N_PAD, D), jnp.float32),
    mesh=_mesh,
    scratch_types=[
        pltpu.VMEM((NCHM, K), jnp.int32),
        pltpu.VMEM((NCHM, K), jnp.int32),
        pltpu.VMEM((K, D), jnp.float32),
        pltpu.VMEM_SHARED((N_PAD, D), jnp.float32),
        pltpu.SemaphoreType.DMA,
    ],
)
def _spmm_kernel(h_hbm, src_hbm, dst_hbm, out_hbm,
                 idx_s, idx_d, gbuf, acc, sem):
    c = lax.axis_index("c")
    s = lax.axis_index("s")
    cnt = jnp.where(c == 0, NCH0, NCH1)
    rows_per_tile = N_PAD // NS

    pltpu.sync_copy(src_hbm.at[c, s], idx_s)
    pltpu.sync_copy(dst_hbm.at[c, s], idx_d)

    zeros = jnp.zeros((16,), jnp.float32)

    def z_body(i, _):
        gbuf[i // (D // 16), pl.ds((i % (D // 16)) * 16, 16)] = zeros
        return 0

    lax.fori_loop(0, K * D // 16, z_body, 0)

    def zc_body(i, _):
        pltpu.sync_copy(gbuf, acc.at[pl.ds(s * rows_per_tile + i * K, K)])
        return 0

    lax.fori_loop(0, rows_per_tile // K, zc_body, 0)
    plsc.subcore_barrier()

    def body(j, _):
        pltpu.async_copy(h_hbm.at[idx_s.at[j]], gbuf, sem).wait()
        pltpu.sync_copy(gbuf, acc.at[idx_d.at[j]], add=True)
        return 0

    lax.fori_loop(0, cnt, body, 0)
    plsc.subcore_barrier()

    pltpu.sync_copy(
        acc.at[pl.ds(s * rows_per_tile, rows_per_tile)],
        out_hbm.at[c, pl.ds(s * rows_per_tile, rows_per_tile)],
    )


def _p0a_body(x_ref, wfc_ref, bfc_ref, g_ref):
    g_ref[...] = jnp.dot(x_ref[...], wfc_ref[...],
                         preferred_element_type=jnp.float32) + bfc_ref[...]


def _p0a(x, wfc, bfc):
    return pl.pallas_call(
        _p0a_body,
        grid=(GRID,),
        in_specs=[
            pl.BlockSpec((BN, D), lambda i: (i, 0)),
            pl.BlockSpec((D, D), lambda i: (0, 0)),
            pl.BlockSpec((1, D), lambda i: (0, 0)),
        ],
        out_specs=pl.BlockSpec((BN, D), lambda i: (i, 0)),
        out_shape=jax.ShapeDtypeStruct((N_PAD, D), jnp.float32),
    )(x, wfc, bfc)


def _p0b_body(g_ref, degs_ref, degd_ref, h1_ref, ns_ref, nd_ref):
    i = pl.program_id(0)
    ones2 = jnp.ones((NC, 1), jnp.float32)
    ds_col = lax.dot_general(degs_ref[...], ones2, (((0,), (0,)), ((), ())))
    dd_col = lax.dot_general(degd_ref[...], ones2, (((0,), (0,)), ((), ())))
    row = lax.broadcasted_iota(jnp.int32, (BN, 1), 0) + i * BN
    valid = row < N
    ns = jnp.where(valid, lax.rsqrt(jnp.clip(ds_col, 1.0, None)), 0.0)
    nd = jnp.where(valid, lax.rsqrt(jnp.clip(dd_col, 1.0, None)), 0.0)
    h1_ref[...] = g_ref[...] * ns
    ns_ref[...] = ns
    nd_ref[...] = nd


def _p0b(g, degs, degd):
    return pl.pallas_call(
        _p0b_body,
        grid=(GRID,),
        in_specs=[
            pl.BlockSpec((BN, D), lambda i: (i, 0)),
            pl.BlockSpec((NC, BN), lambda i: (0, i)),
            pl.BlockSpec((NC, BN), lambda i: (0, i)),
        ],
        out_specs=[
            pl.BlockSpec((BN, D), lambda i: (i, 0)),
            pl.BlockSpec((BN, 1), lambda i: (i, 0)),
            pl.BlockSpec((BN, 1), lambda i: (i, 0)),
        ],
        out_shape=[
            jax.ShapeDtypeStruct((N_PAD, D), jnp.float32),
            jax.ShapeDtypeStruct((N_PAD, 1), jnp.float32),
            jax.ShapeDtypeStruct((N_PAD, 1), jnp.float32),
        ],
    )(g, degs, degd)


def _leaky(x):
    return jnp.where(x >= 0, x, 0.01 * x)


def _pg_body(parts_ref, nd_ref, ns_ref, prev_ref, w_ref, b_ref, wg_ref, bg_ref,
             g_out_ref, h_out_ref):
    agg = (parts_ref[0] + parts_ref[1]) * nd_ref[...]
    x = _leaky(jnp.dot(agg, w_ref[...],
                       preferred_element_type=jnp.float32) + b_ref[...])
    t = jnp.dot(x, wg_ref[...], preferred_element_type=jnp.float32) + bg_ref[...]
    g = x * t + prev_ref[...] * (1.0 - t)
    g_out_ref[...] = g
    h_out_ref[...] = g * ns_ref[...]


def _pg(parts, nd, ns, prev, w, b, wg, bg):
    return pl.pallas_call(
        _pg_body,
        grid=(GRID,),
        in_specs=[
            pl.BlockSpec((NC, BN, D), lambda i: (0, i, 0)),
            pl.BlockSpec((BN, 1), lambda i: (i, 0)),
            pl.BlockSpec((BN, 1), lambda i: (i, 0)),
            pl.BlockSpec((BN, D), lambda i: (i, 0)),
            pl.BlockSpec((D, D), lambda i: (0, 0)),
            pl.BlockSpec((1, D), lambda i: (0, 0)),
            pl.BlockSpec((D, 1), lambda i: (0, 0)),
            pl.BlockSpec((1, 1), lambda i: (0, 0)),
        ],
        out_specs=[
            pl.BlockSpec((BN, D), lambda i: (i, 0)),
            pl.BlockSpec((BN, D), lambda i: (i, 0)),
        ],
        out_shape=[
            jax.ShapeDtypeStruct((N_PAD, D), jnp.float32),
            jax.ShapeDtypeStruct((N_PAD, D), jnp.float32),
        ],
    )(parts, nd, ns, prev, w, b, wg, bg)


def _p3_body(parts_ref, nd_ref, w_ref, b_ref, out_ref):
    agg = (parts_ref[0] + parts_ref[1]) * nd_ref[...]
    out_ref[...] = _leaky(
        jnp.dot(agg, w_ref[...],
                preferred_element_type=jnp.float32) + b_ref[...])[:, :C]


BN3 = 400


def _p3(parts, nd, w3p, b3p):
    return pl.pallas_call(
        _p3_body,
        grid=(N // BN3,),
        in_specs=[
            pl.BlockSpec((NC, BN3, D), lambda i: (0, i, 0)),
            pl.BlockSpec((BN3, 1), lambda i: (i, 0)),
            pl.BlockSpec((D, 256), lambda i: (0, 0)),
            pl.BlockSpec((1, 256), lambda i: (0, 0)),
        ],
        out_specs=pl.BlockSpec((BN3, C), lambda i: (i, 0)),
        out_shape=jax.ShapeDtypeStruct((N, C), jnp.float32),
    )(parts, nd, w3p, b3p)


def kernel(g_init_emb, edge_index, Wfc, bfc, W1, W2, b2, W3, b3, Wg1, bg1, Wg2, bg2):
    src = edge_index[0]
    dst = edge_index[1]

    def pack(e):
        e = jnp.concatenate([e, jnp.full((E_PAD - E,), DUMMY, jnp.int32)])
        e0 = e[: NS * NCH0 * K].reshape(1, NS, NCH0, K)
        e1 = e[NS * NCH0 * K:].reshape(1, NS, NCH1, K)
        e1 = jnp.pad(e1, ((0, 0), (0, 0), (0, NCHM - NCH1), (0, 0)),
                     constant_values=DUMMY)
        return jnp.concatenate([e0, e1], axis=0)

    src_r = pack(src)
    dst_r = pack(dst)

    x = jnp.pad(g_init_emb, ((0, N_PAD - N), (0, 0)))
    w3p = jnp.pad(W3, ((0, 0), (0, 256 - C)))
    b3p = jnp.pad(b3, ((0, 256 - C),)).reshape(1, 256)

    degs, degd = _deg_kernel(src_r, dst_r)
    g_feat = _p0a(x, Wfc, bfc.reshape(1, D))
    h1, ns, nd = _p0b(g_feat, degs, degd)

    parts1 = _spmm_kernel(h1, src_r, dst_r)
    g1, h2 = _pg(parts1, nd, ns, g_feat, W1, jnp.zeros((1, D), jnp.float32),
                 Wg1, bg1.reshape(1, 1))

    parts2 = _spmm_kernel(h2, src_r, dst_r)
    g2, h3 = _pg(parts2, nd, ns, g1, W2, b2.reshape(1, D), Wg2, bg2.reshape(1, 1))

    parts3 = _spmm_kernel(h3, src_r, dst_r)
    return _p3(parts3, nd, w3p, b3p)


# speedup vs baseline: 1.5766x; 1.0024x over previous
import functools

import jax
import jax.numpy as jnp
from jax import lax
from jax.experimental import pallas as pl
from jax.experimental.pallas import tpu as pltpu
from jax.experimental.pallas import tpu_sc as plsc

N = 10000
E = 320000
D = 128
C = 129

NC = 2
NS = 16
NW = NC * NS

N_PAD = 10240
DUMMY = N
K = 128
NCH0 = 112
NCH1 = 46
NCHM = NCH0
E_PAD = NS * K * (NCH0 + NCH1)

BN = 1024
GRID = N_PAD // BN

_mesh = plsc.VectorSubcoreMesh(core_axis_name="c", subcore_axis_name="s")


@functools.partial(
    pl.kernel,
    out_type=[
        jax.ShapeDtypeStruct((NC, N_PAD), jnp.float32),
        jax.ShapeDtypeStruct((NC, N_PAD), jnp.float32),
    ],
    mesh=_mesh,
    scratch_types=[
        pltpu.VMEM((NCHM, K), jnp.int32),
        pltpu.VMEM((NCHM, K), jnp.int32),
        pltpu.VMEM((K,), jnp.float32),
        pltpu.VMEM((N_PAD // NS,), jnp.float32),
        pltpu.VMEM_SHARED((N_PAD,), jnp.float32),
        pltpu.VMEM_SHARED((N_PAD,), jnp.float32),
    ],
)
def _deg_kernel(src_hbm, dst_hbm, outs_hbm, outd_hbm,
                idx_s, idx_d, ones_v, zwin, acc_s, acc_d):
    c = lax.axis_index("c")
    s = lax.axis_index("s")
    cnt = jnp.where(c == 0, NCH0, NCH1)
    rpt = N_PAD // NS

    pltpu.sync_copy(src_hbm.at[c, s], idx_s)
    pltpu.sync_copy(dst_hbm.at[c, s], idx_d)

    ones = jnp.ones((16,), jnp.float32)
    zeros = jnp.zeros((16,), jnp.float32)

    def o_body(i, _):
        ones_v[pl.ds(i * 16, 16)] = ones
        return 0

    lax.fori_loop(0, K // 16, o_body, 0)

    def z_body(i, _):
        zwin[pl.ds(i * 16, 16)] = zeros
        return 0

    lax.fori_loop(0, rpt // 16, z_body, 0)
    pltpu.sync_copy(zwin, acc_s.at[pl.ds(s * rpt, rpt)])
    pltpu.sync_copy(zwin, acc_d.at[pl.ds(s * rpt, rpt)])
    plsc.subcore_barrier()

    def e_body(j, _):
        pltpu.sync_copy(ones_v, acc_s.at[idx_s.at[j]], add=True)
        pltpu.sync_copy(ones_v, acc_d.at[idx_d.at[j]], add=True)
        return 0

    lax.fori_loop(0, cnt, e_body, 0)
    plsc.subcore_barrier()

    pltpu.sync_copy(acc_s.at[pl.ds(s * rpt, rpt)], outs_hbm.at[c, pl.ds(s * rpt, rpt)])
    pltpu.sync_copy(acc_d.at[pl.ds(s * rpt, rpt)], outd_hbm.at[c, pl.ds(s * rpt, rpt)])


@functools.partial(
    pl.kernel,
    out_type=jax.ShapeDtypeStruct((NC, N_PAD, D), jnp.float32),
    mesh=_mesh,
    scratch_types=[
        pltpu.VMEM((NCHM, K), jnp.int32),
        pltpu.VMEM((NCHM, K), jnp.int32),
        pltpu.VMEM((K, D), jnp.float32),
        pltpu.VMEM_SHARED((N_PAD, D), jnp.float32),
        pltpu.SemaphoreType.DMA,
    ],
)
def _spmm_kernel(h_hbm, src_hbm, dst_hbm, out_hbm,
                 idx_s, idx_d, gbuf, acc, sem):
    c = lax.axis_index("c")
    s = lax.axis_index("s")
    cnt = jnp.where(c == 0, NCH0, NCH1)
    rows_per_tile = N_PAD // NS

    pltpu.sync_copy(src_hbm.at[c, s], idx_s)
    pltpu.sync_copy(dst_hbm.at[c, s], idx_d)

    zeros = jnp.zeros((16,), jnp.float32)

    def z_body(i, _):
        gbuf[i // (D // 16), pl.ds((i % (D // 16)) * 16, 16)] = zeros
        return 0

    lax.fori_loop(0, K * D // 16, z_body, 0)

    def zc_body(i, _):
        pltpu.sync_copy(gbuf, acc.at[pl.ds(s * rows_per_tile + i * K, K)])
        return 0

    lax.fori_loop(0, rows_per_tile // K, zc_body, 0)
    plsc.subcore_barrier()

    def body(j, _):
        pltpu.async_copy(h_hbm.at[idx_s.at[j]], gbuf, sem).wait()
        pltpu.sync_copy(gbuf, acc.at[idx_d.at[j]], add=True)
        return 0

    lax.fori_loop(0, cnt, body, 0)
    plsc.subcore_barrier()

    pltpu.sync_copy(
        acc.at[pl.ds(s * rows_per_tile, rows_per_tile)],
        out_hbm.at[c, pl.ds(s * rows_per_tile, rows_per_tile)],
    )


def _p0a_body(x_ref, wfc_ref, bfc_ref, g_ref):
    g_ref[...] = jnp.dot(x_ref[...], wfc_ref[...],
                         preferred_element_type=jnp.float32) + bfc_ref[...]


def _p0a(x, wfc, bfc):
    return pl.pallas_call(
        _p0a_body,
        grid=(GRID,),
        in_specs=[
            pl.BlockSpec((BN, D), lambda i: (i, 0)),
            pl.BlockSpec((D, D), lambda i: (0, 0)),
            pl.BlockSpec((1, D), lambda i: (0, 0)),
        ],
        out_specs=pl.BlockSpec((BN, D), lambda i: (i, 0)),
        out_shape=jax.ShapeDtypeStruct((N_PAD, D), jnp.float32),
    )(x, wfc, bfc)


def _p0b_body(g_ref, degs_ref, degd_ref, h1_ref, ns_ref, nd_ref):
    i = pl.program_id(0)
    ones2 = jnp.ones((NC, 1), jnp.float32)
    ds_col = lax.dot_general(degs_ref[...], ones2, (((0,), (0,)), ((), ())))
    dd_col = lax.dot_general(degd_ref[...], ones2, (((0,), (0,)), ((), ())))
    row = lax.broadcasted_iota(jnp.int32, (BN, 1), 0) + i * BN
    valid = row < N
    ns = jnp.where(valid, lax.rsqrt(jnp.clip(ds_col, 1.0, None)), 0.0)
    nd = jnp.where(valid, lax.rsqrt(jnp.clip(dd_col, 1.0, None)), 0.0)
    h1_ref[...] = g_ref[...] * ns
    ns_ref[...] = ns
    nd_ref[...] = nd


def _p0b(g, degs, degd):
    return pl.pallas_call(
        _p0b_body,
        grid=(GRID,),
        in_specs=[
            pl.BlockSpec((BN, D), lambda i: (i, 0)),
            pl.BlockSpec((NC, BN), lambda i: (0, i)),
            pl.BlockSpec((NC, BN), lambda i: (0, i)),
        ],
        out_specs=[
            pl.BlockSpec((BN, D), lambda i: (i, 0)),
            pl.BlockSpec((BN, 1), lambda i: (i, 0)),
            pl.BlockSpec((BN, 1), lambda i: (i, 0)),
        ],
        out_shape=[
            jax.ShapeDtypeStruct((N_PAD, D), jnp.float32),
            jax.ShapeDtypeStruct((N_PAD, 1), jnp.float32),
            jax.ShapeDtypeStruct((N_PAD, 1), jnp.float32),
        ],
    )(g, degs, degd)


def _leaky(x):
    return jnp.where(x >= 0, x, 0.01 * x)


def _pg_body(parts_ref, nd_ref, ns_ref, prev_ref, w_ref, b_ref, wg_ref, bg_ref,
             g_out_ref, h_out_ref):
    agg = (parts_ref[0] + parts_ref[1]) * nd_ref[...]
    x = _leaky(jnp.dot(agg, w_ref[...],
                       preferred_element_type=jnp.float32) + b_ref[...])
    t = jnp.dot(x, wg_ref[...], preferred_element_type=jnp.float32) + bg_ref[...]
    g = x * t + prev_ref[...] * (1.0 - t)
    g_out_ref[...] = g
    h_out_ref[...] = g * ns_ref[...]


def _pg(parts, nd, ns, prev, w, b, wg, bg):
    return pl.pallas_call(
        _pg_body,
        grid=(GRID,),
        in_specs=[
            pl.BlockSpec((NC, BN, D), lambda i: (0, i, 0)),
            pl.BlockSpec((BN, 1), lambda i: (i, 0)),
            pl.BlockSpec((BN, 1), lambda i: (i, 0)),
            pl.BlockSpec((BN, D), lambda i: (i, 0)),
            pl.BlockSpec((D, D), lambda i: (0, 0)),
            pl.BlockSpec((1, D), lambda i: (0, 0)),
            pl.BlockSpec((D, 1), lambda i: (0, 0)),
            pl.BlockSpec((1, 1), lambda i: (0, 0)),
        ],
        out_specs=[
            pl.BlockSpec((BN, D), lambda i: (i, 0)),
            pl.BlockSpec((BN, D), lambda i: (i, 0)),
        ],
        out_shape=[
            jax.ShapeDtypeStruct((N_PAD, D), jnp.float32),
            jax.ShapeDtypeStruct((N_PAD, D), jnp.float32),
        ],
    )(parts, nd, ns, prev, w, b, wg, bg)


def _p3_body(parts_ref, nd_ref, w_ref, b_ref, out_ref):
    agg = (parts_ref[0] + parts_ref[1]) * nd_ref[...]
    out_ref[...] = _leaky(
        jnp.dot(agg, w_ref[...],
                preferred_element_type=jnp.float32) + b_ref[...])[:, :C]


BN3 = 400


def _p3(parts, nd, w3p, b3p):
    return pl.pallas_call(
        _p3_body,
        grid=(N // BN3,),
        in_specs=[
            pl.BlockSpec((NC, BN3, D), lambda i: (0, i, 0)),
            pl.BlockSpec((BN3, 1), lambda i: (i, 0)),
            pl.BlockSpec((D, 256), lambda i: (0, 0)),
            pl.BlockSpec((1, 256), lambda i: (0, 0)),
        ],
        out_specs=pl.BlockSpec((BN3, C), lambda i: (i, 0)),
        out_shape=jax.ShapeDtypeStruct((N, C), jnp.float32),
    )(parts, nd, w3p, b3p)


def kernel(g_init_emb, edge_index, Wfc, bfc, W1, W2, b2, W3, b3, Wg1, bg1, Wg2, bg2):
    src = edge_index[0]
    dst = edge_index[1]

    def pack(e):
        e = jnp.concatenate([e, jnp.full((E_PAD - E,), DUMMY, jnp.int32)])
        e0 = e[: NS * NCH0 * K].reshape(1, NS, NCH0, K)
        e1 = e[NS * NCH0 * K:].reshape(1, NS, NCH1, K)
        e1 = jnp.pad(e1, ((0, 0), (0, 0), (0, NCHM - NCH1), (0, 0)),
                     constant_values=DUMMY)
        return jnp.concatenate([e0, e1], axis=0)

    src_r = pack(src)
    dst_r = pack(dst)

    x = jnp.pad(g_init_emb, ((0, N_PAD - N), (0, 0)))
    w3p = jnp.pad(W3, ((0, 0), (0, 256 - C)))
    b3p = jnp.pad(b3, ((0, 256 - C),)).reshape(1, 256)

    degs, degd = _deg_kernel(src_r, dst_r)
    g_feat = _p0a(x, Wfc, bfc.reshape(1, D))
    h1, ns, nd = _p0b(g_feat, degs, degd)

    parts1 = _spmm_kernel(h1, src_r, dst_r)
    g1, h2 = _pg(parts1, nd, ns, g_feat, W1, jnp.zeros((1, D), jnp.float32),
                 Wg1, bg1.reshape(1, 1))

    parts2 = _spmm_kernel(h2, src_r, dst_r)
    g2, h3 = _pg(parts2, nd, ns, g1, W2, b2.reshape(1, D), Wg2, bg2.reshape(1, 1))

    parts3 = _spmm_kernel(h3, src_r, dst_r)
    return _p3(parts3, nd, w3p, b3p)
